# Initial kernel scaffold; baseline (speedup 1.0000x reference)
import jax
import jax.numpy as jnp
from jax.experimental import pallas as pl


def kernel(relate_src, relate_dst, similar_src, similar_dst, embed_drug, embed_side, W, b):
    raise NotImplementedError("write your pallas kernel here")


# speedup vs baseline: 5.0772x; 5.0772x over previous
import functools

import jax
import jax.numpy as jnp
from jax import lax
from jax.experimental import pallas as pl
from jax.experimental.pallas import tpu as pltpu
from jax.experimental.pallas import tpu_sc as plsc

ND = 10000
D = 128
E = 320000

NC = 2
NSUB = 16
NW = NC * NSUB
EW = E // NW
CD = 400
C = 200
RPS = 624
ZR = 104

_mesh = plsc.VectorSubcoreMesh(core_axis_name="c", subcore_axis_name="s")


@functools.partial(
    pl.kernel,
    out_type=jax.ShapeDtypeStruct((NC * 4 * ND,), jnp.float32),
    mesh=_mesh,
    scratch_types=[
        pltpu.VMEM((CD,), jnp.int32),
        pltpu.VMEM((CD,), jnp.float32),
        pltpu.VMEM((624,), jnp.float32),
        pltpu.VMEM((624,), jnp.float32),
        pltpu.VMEM_SHARED((ND,), jnp.float32),
    ],
)
def _deg_call(rs, rd, ss, sd, out, idx_v, ones_v, zer_v, tmp_v, acc):
    cid = lax.axis_index("c")
    sid = lax.axis_index("s")
    wid = sid * NC + cid
    ones16 = jnp.ones((16,), jnp.float32)
    zer16 = jnp.zeros((16,), jnp.float32)
    for i in range(CD // 16):
        ones_v[pl.ds(i * 16, 16)] = ones16
    for i in range(624 // 16):
        zer_v[pl.ds(i * 16, 16)] = zer16

    for r, eref in enumerate((rs, rd, ss, sd)):
        pltpu.sync_copy(zer_v, acc.at[pl.ds(sid * 624, 624)])

        @pl.when(sid == 0)
        def _():
            pltpu.sync_copy(zer_v.at[pl.ds(0, 16)], acc.at[pl.ds(9984, 16)])

        plsc.subcore_barrier()

        def body(i, carry):
            base = wid * EW + i * CD
            pltpu.sync_copy(eref.at[pl.ds(base, CD)], idx_v)
            pltpu.sync_copy(ones_v, acc.at[idx_v], add=True)
            return carry

        lax.fori_loop(0, EW // CD, body, 0)
        plsc.subcore_barrier()
        obase = (cid * 4 + r) * ND
        pltpu.sync_copy(acc.at[pl.ds(sid * 624, 624)], tmp_v)
        pltpu.sync_copy(tmp_v, out.at[pl.ds(obase + sid * 624, 624)])

        @pl.when(sid == 0)
        def _():
            pltpu.sync_copy(acc.at[pl.ds(9984, 16)], tmp_v.at[pl.ds(0, 16)])
            pltpu.sync_copy(tmp_v.at[pl.ds(0, 16)],
                            out.at[pl.ds(obase + 9984, 16)])

        plsc.subcore_barrier()


@functools.partial(
    pl.kernel,
    out_type=jax.ShapeDtypeStruct((4, NC, ND, D), jnp.float32),
    mesh=_mesh,
    scratch_types=[
        pltpu.VMEM((C,), jnp.int32),
        pltpu.VMEM((C,), jnp.int32),
        pltpu.VMEM((C, D), jnp.float32),
        pltpu.VMEM((ZR, D), jnp.float32),
        pltpu.VMEM_SHARED((ND, D), jnp.float32),
        pltpu.SemaphoreType.DMA,
    ],
)
def _agg_call(z0, z1, z2, z3, rs, rd, ss, sd, out,
              si_v, di_v, rows_v, zer_v, acc, sem):
    cid = lax.axis_index("c")
    sid = lax.axis_index("s")
    wid = sid * NC + cid
    zer16 = jnp.zeros((16,), jnp.float32)

    def zbody(i, carry):
        for j in range(D // 16):
            zer_v[i, pl.ds(j * 16, 16)] = zer16
        return carry

    lax.fori_loop(0, ZR, zbody, 0)

    for r, (zr, sref, dref) in enumerate(
            ((z0, rs, rd), (z1, rd, rs), (z2, ss, sd), (z3, sd, ss))):
        for j in range(RPS // ZR):
            pltpu.sync_copy(zer_v, acc.at[pl.ds(sid * RPS + j * ZR, ZR)])

        @pl.when(sid == NSUB - 1)
        def _():
            pltpu.sync_copy(zer_v.at[pl.ds(0, 16)], acc.at[pl.ds(9984, 16)])

        plsc.subcore_barrier()

        def body(i, carry):
            base = wid * EW + i * C
            pltpu.sync_copy(sref.at[pl.ds(base, C)], si_v)
            pltpu.sync_copy(dref.at[pl.ds(base, C)], di_v)
            pltpu.async_copy(zr.at[si_v], rows_v, sem).wait()
            pltpu.sync_copy(rows_v, acc.at[di_v], add=True)
            return carry

        lax.fori_loop(0, EW // C, body, 0)
        plsc.subcore_barrier()
        bounce = rows_v.at[pl.ds(0, ZR)]
        for j in range(RPS // ZR):
            pltpu.sync_copy(acc.at[pl.ds(sid * RPS + j * ZR, ZR)], bounce)
            pltpu.sync_copy(bounce, out.at[r, cid, pl.ds(sid * RPS + j * ZR, ZR)])

        @pl.when(sid == NSUB - 1)
        def _():
            pltpu.sync_copy(acc.at[pl.ds(9984, 16)], rows_v.at[pl.ds(0, 16)])
            pltpu.sync_copy(rows_v.at[pl.ds(0, 16)],
                            out.at[r, cid, pl.ds(9984, 16)])

        plsc.subcore_barrier()


_B = 1000


def _scales(cnt):
    deg = jnp.maximum(cnt[:, 0:4] + cnt[:, 4:8], 1.0)
    return lax.rsqrt(deg)


def _mm4(hd, hs, sc, w, z_refs):
    prec = jax.lax.Precision.HIGHEST
    z_refs[0][...] = jnp.dot(hd * sc[:, 0:1], w[0], precision=prec)
    z_refs[1][...] = jnp.dot(hs * sc[:, 1:2], w[1], precision=prec)
    z_refs[2][...] = jnp.dot(hd * sc[:, 2:3], w[2], precision=prec)
    z_refs[3][...] = jnp.dot(hd * sc[:, 3:4], w[3], precision=prec)


def _tc_first_body(hd_ref, hs_ref, cnt_ref, w_ref,
                   z0_ref, z1_ref, z2_ref, z3_ref):
    sc = _scales(cnt_ref[...])
    _mm4(hd_ref[...], hs_ref[...], sc, w_ref[...],
         (z0_ref, z1_ref, z2_ref, z3_ref))


def _combine(p, sc, b):
    ps = p[:, 0] + p[:, 1]
    hs = sc[:, 1:2] * ps[0] + b[0][None, :]
    hd = (sc[:, 0:1] * ps[1] + sc[:, 3:4] * ps[2] + sc[:, 2:3] * ps[3]
          + (b[1] + b[2] + b[3])[None, :])
    return hd, hs


def _tc_mid_body(p_ref, cnt_ref, b_ref, w_ref,
                 z0_ref, z1_ref, z2_ref, z3_ref):
    sc = _scales(cnt_ref[...])
    hd, hs = _combine(p_ref[...], sc, b_ref[...])
    hd = jnp.maximum(hd, 0.0)
    hs = jnp.maximum(hs, 0.0)
    _mm4(hd, hs, sc, w_ref[...], (z0_ref, z1_ref, z2_ref, z3_ref))


def _tc_final_body(p_ref, cnt_ref, b_ref, hd_ref, hs_ref):
    sc = _scales(cnt_ref[...])
    hd, hs = _combine(p_ref[...], sc, b_ref[...])
    hd_ref[...] = hd
    hs_ref[...] = hs


_z_struct = tuple(jax.ShapeDtypeStruct((ND, D), jnp.float32) for _ in range(4))
_zspec = pl.BlockSpec((_B, D), lambda i: (i, 0))
_cntspec = pl.BlockSpec((_B, 8), lambda i: (i, 0))
_pspec = pl.BlockSpec((4, NC, _B, D), lambda i: (0, 0, i, 0))
_wspec = pl.BlockSpec((4, D, D), lambda i: (0, 0, 0))
_bspec = pl.BlockSpec((4, D), lambda i: (0, 0))

_tc_first = pl.pallas_call(
    _tc_first_body,
    grid=(ND // _B,),
    in_specs=[_zspec, _zspec, _cntspec, _wspec],
    out_specs=[_zspec] * 4,
    out_shape=list(_z_struct),
)

_tc_mid = pl.pallas_call(
    _tc_mid_body,
    grid=(ND // _B,),
    in_specs=[_pspec, _cntspec, _bspec, _wspec],
    out_specs=[_zspec] * 4,
    out_shape=list(_z_struct),
)

_tc_final = pl.pallas_call(
    _tc_final_body,
    grid=(ND // _B,),
    in_specs=[_pspec, _cntspec, _bspec],
    out_specs=[_zspec] * 2,
    out_shape=list(_z_struct[:2]),
)


def kernel(relate_src, relate_dst, similar_src, similar_dst,
           embed_drug, embed_side, W, b):
    rs = relate_src.astype(jnp.int32)
    rd = relate_dst.astype(jnp.int32)
    ss = similar_src.astype(jnp.int32)
    sd = similar_dst.astype(jnp.int32)

    counts = _deg_call(rs, rd, ss, sd).reshape(NC, 4, ND)
    cnt_t = counts.transpose(2, 0, 1).reshape(ND, 8)

    z = _tc_first(embed_drug, embed_side, cnt_t, W[0])
    p = _agg_call(z[0], z[1], z[2], z[3], rs, rd, ss, sd)

    z = _tc_mid(p, cnt_t, b[0], W[1])
    p = _agg_call(z[0], z[1], z[2], z[3], rs, rd, ss, sd)

    z = _tc_mid(p, cnt_t, b[1], W[2])
    p = _agg_call(z[0], z[1], z[2], z[3], rs, rd, ss, sd)

    hd, hs = _tc_final(p, cnt_t, b[2])
    return (hd, hs)
